# Initial kernel scaffold; baseline (speedup 1.0000x reference)
import jax
import jax.numpy as jnp
from jax.experimental import pallas as pl


def kernel(x, edge_index, W, b, Wd, bd):
    raise NotImplementedError("write your pallas kernel here")


# speedup vs baseline: 87.9405x; 87.9405x over previous
import functools

import jax
import jax.numpy as jnp
from jax import lax
from jax.experimental import pallas as pl
from jax.experimental.pallas import tpu as pltpu
from jax.experimental.pallas import tpu_sc as plsc

LANES = 16
NC = 2
NS = 16


def _sc_edge_weights(edge_index, n_nodes):
    E = edge_index.shape[1]
    NW = NC * NS
    EPT = E // NS
    EPH = E // NW
    P1 = EPT // LANES
    P2 = EPH // LANES
    NPAD = -(-n_nodes // (NS * LANES)) * (NS * LANES)
    NV = NPAD // LANES
    STRIPE = NPAD // NS
    SV = STRIPE // LANES

    mesh = plsc.VectorSubcoreMesh(core_axis_name="c", subcore_axis_name="s")

    @functools.partial(
        pl.kernel,
        out_type=jax.ShapeDtypeStruct((NW, NPAD), jnp.float32),
        mesh=mesh,
        scratch_types=[
            pltpu.VMEM((EPT,), jnp.int32),
            pltpu.VMEM((EPT,), jnp.int32),
            pltpu.VMEM((NPAD,), jnp.float32),
            pltpu.VMEM((NPAD,), jnp.float32),
            pltpu.VMEM((NPAD,), jnp.float32),
            pltpu.VMEM((STRIPE,), jnp.float32),
            pltpu.VMEM((STRIPE,), jnp.float32),
            pltpu.VMEM_SHARED((NS, 2, NPAD), jnp.float32),
            pltpu.VMEM_SHARED((2, NPAD), jnp.float32),
        ],
        compiler_params=pltpu.CompilerParams(needs_layout_passes=False),
    )
    def ew_kernel(edge_hbm, wp_hbm, src_v, dst_v, degs_v, degd_v, w_v,
                  sacc_v, stmp_v, stage_sh, red_sh):
        c = lax.axis_index("c")
        t = lax.axis_index("s")
        zeros = jnp.zeros((LANES,), jnp.float32)
        ones = jnp.full((LANES,), 1.0, jnp.float32)

        pltpu.sync_copy(edge_hbm.at[pl.ds(t * EPT, EPT)], src_v)
        pltpu.sync_copy(edge_hbm.at[pl.ds(E + t * EPT, EPT)], dst_v)

        def zero_body(i, _):
            sl = pl.ds(i * LANES, LANES)
            degs_v[sl] = zeros
            degd_v[sl] = zeros
            w_v[sl] = zeros
            return 0

        lax.fori_loop(0, NV, zero_body, 0)

        def p1_body(i, _):
            sl = pl.ds(i * LANES, LANES)
            plsc.addupdate_scatter(degs_v, [src_v[sl]], ones)
            plsc.addupdate_scatter(degd_v, [dst_v[sl]], ones)
            return 0

        lax.fori_loop(0, P1, p1_body, 0)

        pltpu.sync_copy(degs_v, stage_sh.at[t, 0])
        pltpu.sync_copy(degd_v, stage_sh.at[t, 1])
        plsc.subcore_barrier()

        for a in range(2):
            pltpu.sync_copy(stage_sh.at[0, a, pl.ds(t * STRIPE, STRIPE)], sacc_v)

            def red_body(tt, _, a=a):
                pltpu.sync_copy(
                    stage_sh.at[tt, a, pl.ds(t * STRIPE, STRIPE)], stmp_v)

                def add_body(j, _):
                    sl = pl.ds(j * LANES, LANES)
                    sacc_v[sl] = sacc_v[sl] + stmp_v[sl]
                    return 0

                lax.fori_loop(0, SV, add_body, 0)
                return 0

            lax.fori_loop(1, NS, red_body, 0)
            pltpu.sync_copy(sacc_v, red_sh.at[a, pl.ds(t * STRIPE, STRIPE)])
        plsc.subcore_barrier()

        pltpu.sync_copy(red_sh.at[0], degs_v)
        pltpu.sync_copy(red_sh.at[1], degd_v)

        magic = jnp.full((LANES,), 0x5F3759DF, jnp.int32)
        half = jnp.full((LANES,), 0.5, jnp.float32)
        th = jnp.full((LANES,), 1.5, jnp.float32)

        def rsqrt16(v):
            y = plsc.bitcast(magic - (plsc.bitcast(v, jnp.int32) >> 1),
                             jnp.float32)
            y = y * (th - half * v * y * y)
            y = y * (th - half * v * y * y)
            y = y * (th - half * v * y * y)
            return y

        def rs_body(i, _):
            sl = pl.ds(i * LANES, LANES)
            degs_v[sl] = rsqrt16(jnp.maximum(degs_v[sl], ones))
            degd_v[sl] = rsqrt16(jnp.maximum(degd_v[sl], ones))
            return 0

        lax.fori_loop(0, NV, rs_body, 0)

        base = c * EPH

        def p2_body(i, _):
            sl = pl.ds(base + i * LANES, LANES)
            rd = plsc.load_gather(degd_v, [dst_v[sl]])
            plsc.addupdate_scatter(w_v, [src_v[sl]], rd)
            return 0

        lax.fori_loop(0, P2, p2_body, 0)

        def fin_body(i, _):
            sl = pl.ds(i * LANES, LANES)
            w_v[sl] = w_v[sl] * degs_v[sl]
            return 0

        lax.fori_loop(0, NV, fin_body, 0)

        pltpu.sync_copy(w_v, wp_hbm.at[c * NS + t])

    return ew_kernel(edge_index.reshape(-1))


def _tc_head(wp, x, W, b, Wd, bd):
    N, D = x.shape
    L = Wd.shape[1]

    def body(wp_ref, x_ref, W_ref, b_ref, Wd_ref, bd_ref, o_ref):
        w = jnp.sum(wp_ref[...], axis=0, keepdims=True)[:, :N]
        t = jnp.sum(w)
        dn = (((1,), (0,)), ((), ()))
        s = lax.dot_general(w, x_ref[...], dn,
                            preferred_element_type=jnp.float32)
        pooled = lax.dot_general(s, W_ref[...], dn,
                                 preferred_element_type=jnp.float32)
        pooled = pooled + t * b_ref[...]
        logits = lax.dot_general(pooled, Wd_ref[...], dn,
                                 preferred_element_type=jnp.float32)
        logits = logits + bd_ref[...]
        e = jnp.exp(logits - jnp.max(logits))
        o_ref[...] = e / jnp.sum(e)

    return pl.pallas_call(
        body,
        out_shape=jax.ShapeDtypeStruct((1, L), jnp.float32),
    )(wp, x, W, b.reshape(1, D), Wd, bd.reshape(1, L))


def kernel(x, edge_index, W, b, Wd, bd):
    wp = _sc_edge_weights(edge_index, x.shape[0])
    out = _tc_head(wp, x, W, b, Wd, bd)
    return out.reshape(-1)


# speedup vs baseline: 93.5458x; 1.0637x over previous
import functools

import jax
import jax.numpy as jnp
from jax import lax
from jax.experimental import pallas as pl
from jax.experimental.pallas import tpu as pltpu
from jax.experimental.pallas import tpu_sc as plsc

LANES = 16
NC = 2
NS = 16


def _sc_edge_weights(edge_index, n_nodes):
    E = edge_index.shape[1]
    NW = NC * NS
    EPT = E // NS
    EPH = E // NW
    P1 = EPT // LANES
    P2 = EPH // LANES
    NPAD = -(-n_nodes // (NS * LANES)) * (NS * LANES)
    NV = NPAD // LANES
    STRIPE = NPAD // NS
    SV = STRIPE // LANES

    mesh = plsc.VectorSubcoreMesh(core_axis_name="c", subcore_axis_name="s")

    @functools.partial(
        pl.kernel,
        out_type=jax.ShapeDtypeStruct((NW, NPAD), jnp.float32),
        mesh=mesh,
        scratch_types=[
            pltpu.VMEM((EPT,), jnp.int32),
            pltpu.VMEM((EPT,), jnp.int32),
            pltpu.VMEM((NPAD,), jnp.float32),
            pltpu.VMEM((NPAD,), jnp.float32),
            pltpu.VMEM((NPAD,), jnp.float32),
            pltpu.VMEM((NS, 2, STRIPE), jnp.float32),
            pltpu.VMEM((2, STRIPE), jnp.float32),
            pltpu.VMEM_SHARED((NS, 2, NPAD), jnp.float32),
            pltpu.VMEM_SHARED((2, NPAD), jnp.float32),
            pltpu.SemaphoreType.DMA,
        ],
        compiler_params=pltpu.CompilerParams(needs_layout_passes=False),
    )
    def ew_kernel(edge_hbm, wp_hbm, src_v, dst_v, degs_v, degd_v, w_v,
                  peer_v, sred_v, stage_sh, red_sh, sem):
        c = lax.axis_index("c")
        t = lax.axis_index("s")
        zeros = jnp.zeros((LANES,), jnp.float32)
        ones = jnp.full((LANES,), 1.0, jnp.float32)

        ld_s = pltpu.async_copy(edge_hbm.at[pl.ds(t * EPT, EPT)], src_v, sem)
        ld_d = pltpu.async_copy(edge_hbm.at[pl.ds(E + t * EPT, EPT)],
                                dst_v, sem)

        def zero_body(i, _):
            sl = pl.ds(i * LANES, LANES)
            degs_v[sl] = zeros
            degd_v[sl] = zeros
            return 0

        lax.fori_loop(0, NV, zero_body, 0, unroll=8)
        ld_s.wait()
        ld_d.wait()

        def p1_body(i, _):
            sl = pl.ds(i * LANES, LANES)
            plsc.addupdate_scatter(degs_v, [src_v[sl]], ones)
            plsc.addupdate_scatter(degd_v, [dst_v[sl]], ones)
            return 0

        lax.fori_loop(0, P1, p1_body, 0, unroll=8)

        st_s = pltpu.async_copy(degs_v, stage_sh.at[t, 0], sem)
        st_d = pltpu.async_copy(degd_v, stage_sh.at[t, 1], sem)
        st_s.wait()
        st_d.wait()
        plsc.subcore_barrier()

        fetches = []
        for tt in range(NS):
            for a in range(2):
                fetches.append(pltpu.async_copy(
                    stage_sh.at[tt, a, pl.ds(t * STRIPE, STRIPE)],
                    peer_v.at[tt, a], sem))
        for f in fetches:
            f.wait()

        def acc_body(j, _):
            for a in range(2):
                sl = pl.ds(j * LANES, LANES)
                v = peer_v[0, a, sl]
                for tt in range(1, NS):
                    v = v + peer_v[tt, a, sl]
                sred_v[a, sl] = v
            return 0

        lax.fori_loop(0, SV, acc_body, 0, unroll=2)

        wb0 = pltpu.async_copy(sred_v.at[0],
                               red_sh.at[0, pl.ds(t * STRIPE, STRIPE)], sem)
        wb1 = pltpu.async_copy(sred_v.at[1],
                               red_sh.at[1, pl.ds(t * STRIPE, STRIPE)], sem)
        wb0.wait()
        wb1.wait()
        plsc.subcore_barrier()

        rb0 = pltpu.async_copy(red_sh.at[0], degs_v, sem)
        rb1 = pltpu.async_copy(red_sh.at[1], degd_v, sem)
        rb0.wait()
        rb1.wait()

        magic = jnp.full((LANES,), 0x5F3759DF, jnp.int32)
        half = jnp.full((LANES,), 0.5, jnp.float32)
        th = jnp.full((LANES,), 1.5, jnp.float32)

        def rsqrt16(v):
            y = plsc.bitcast(magic - (plsc.bitcast(v, jnp.int32) >> 1),
                             jnp.float32)
            y = y * (th - half * v * y * y)
            y = y * (th - half * v * y * y)
            return y

        def rs_body(i, _):
            sl = pl.ds(i * LANES, LANES)
            degd_v[sl] = rsqrt16(jnp.maximum(degd_v[sl], ones))
            w_v[sl] = zeros
            return 0

        lax.fori_loop(0, NV, rs_body, 0, unroll=4)

        base = c * EPH

        def p2_body(i, _):
            sl = pl.ds(base + i * LANES, LANES)
            rd = plsc.load_gather(degd_v, [dst_v[sl]])
            plsc.addupdate_scatter(w_v, [src_v[sl]], rd)
            return 0

        lax.fori_loop(0, P2, p2_body, 0, unroll=8)

        def fin_body(i, _):
            sl = pl.ds(i * LANES, LANES)
            w_v[sl] = w_v[sl] * rsqrt16(jnp.maximum(degs_v[sl], ones))
            return 0

        lax.fori_loop(0, NV, fin_body, 0, unroll=4)

        pltpu.sync_copy(w_v, wp_hbm.at[c * NS + t])

    return ew_kernel(edge_index.reshape(-1))


def _tc_head(wp, x, W, b, Wd, bd):
    N, D = x.shape
    L = Wd.shape[1]

    def body(wp_ref, x_ref, W_ref, b_ref, Wd_ref, bd_ref, o_ref):
        w = jnp.sum(wp_ref[...], axis=0, keepdims=True)[:, :N]
        t = jnp.sum(w)
        dn = (((1,), (0,)), ((), ()))
        s = lax.dot_general(w, x_ref[...], dn,
                            preferred_element_type=jnp.float32)
        pooled = lax.dot_general(s, W_ref[...], dn,
                                 preferred_element_type=jnp.float32)
        pooled = pooled + t * b_ref[...]
        logits = lax.dot_general(pooled, Wd_ref[...], dn,
                                 preferred_element_type=jnp.float32)
        logits = logits + bd_ref[...]
        e = jnp.exp(logits - jnp.max(logits))
        o_ref[...] = e / jnp.sum(e)

    return pl.pallas_call(
        body,
        out_shape=jax.ShapeDtypeStruct((1, L), jnp.float32),
    )(wp, x, W, b.reshape(1, D), Wd, bd.reshape(1, L))


def kernel(x, edge_index, W, b, Wd, bd):
    wp = _sc_edge_weights(edge_index, x.shape[0])
    out = _tc_head(wp, x, W, b, Wd, bd)
    return out.reshape(-1)


# speedup vs baseline: 172.9659x; 1.8490x over previous
import functools

import jax
import jax.numpy as jnp
from jax import lax
from jax.experimental import pallas as pl
from jax.experimental.pallas import tpu as pltpu
from jax.experimental.pallas import tpu_sc as plsc

LANES = 16
NC = 2
NS = 16


def _sc_edge_weights(edge_index, n_nodes):
    E = edge_index.shape[1]
    NW = NC * NS
    EPT = E // NS
    EPH = E // NW
    P1 = EPT // LANES
    P2 = EPH // LANES
    NPAD = -(-n_nodes // (NS * LANES)) * (NS * LANES)
    NV = NPAD // LANES
    STRIPE = NPAD // NS
    SV = STRIPE // LANES

    mesh = plsc.VectorSubcoreMesh(core_axis_name="c", subcore_axis_name="s")

    @functools.partial(
        pl.kernel,
        out_type=(
            jax.ShapeDtypeStruct((NW, NPAD), jnp.float32),
            jax.ShapeDtypeStruct((NPAD,), jnp.float32),
        ),
        mesh=mesh,
        scratch_types=[
            pltpu.VMEM((EPT,), jnp.int32),
            pltpu.VMEM((EPT,), jnp.int32),
            pltpu.VMEM((NPAD,), jnp.float32),
            pltpu.VMEM((NPAD,), jnp.float32),
            pltpu.VMEM((NPAD,), jnp.float32),
            pltpu.VMEM((NS, 2, STRIPE), jnp.float32),
            pltpu.VMEM((2, STRIPE), jnp.float32),
            pltpu.VMEM_SHARED((NS, 2, NPAD), jnp.float32),
            pltpu.VMEM_SHARED((NPAD,), jnp.float32),
            pltpu.SemaphoreType.DMA,
        ],
        compiler_params=pltpu.CompilerParams(needs_layout_passes=False),
    )
    def ew_kernel(edge_hbm, wp_hbm, degsrc_hbm, src_v, dst_v, degs_v, degd_v,
                  w_v, peer_v, sred_v, stage_sh, rdst_sh, sem):
        c = lax.axis_index("c")
        t = lax.axis_index("s")
        zeros = jnp.zeros((LANES,), jnp.float32)
        ones = jnp.full((LANES,), 1.0, jnp.float32)

        ld_s = pltpu.async_copy(edge_hbm.at[pl.ds(t * EPT, EPT)], src_v, sem)
        ld_d = pltpu.async_copy(edge_hbm.at[pl.ds(E + t * EPT, EPT)],
                                dst_v, sem)

        @plsc.parallel_loop(0, NV, unroll=8)
        def zero_body(i):
            sl = pl.ds(i * LANES, LANES)
            degs_v[sl] = zeros
            degd_v[sl] = zeros
            w_v[sl] = zeros

        ld_s.wait()
        ld_d.wait()

        @plsc.parallel_loop(0, P1, unroll=8)
        def p1_body(i):
            sl = pl.ds(i * LANES, LANES)
            plsc.addupdate_scatter(degs_v, [src_v[sl]], ones)
            plsc.addupdate_scatter(degd_v, [dst_v[sl]], ones)

        st_s = pltpu.async_copy(degs_v, stage_sh.at[t, 0], sem)
        st_d = pltpu.async_copy(degd_v, stage_sh.at[t, 1], sem)
        st_s.wait()
        st_d.wait()
        plsc.subcore_barrier()

        fetches = []
        for tt in range(NS):
            for a in range(2):
                fetches.append(pltpu.async_copy(
                    stage_sh.at[tt, a, pl.ds(t * STRIPE, STRIPE)],
                    peer_v.at[tt, a], sem))
        for f in fetches:
            f.wait()

        @plsc.parallel_loop(0, SV, unroll=4)
        def acc_body(j):
            for a in range(2):
                sl = pl.ds(j * LANES, LANES)
                v = peer_v[0, a, sl]
                for tt in range(1, NS):
                    v = v + peer_v[tt, a, sl]
                sred_v[a, sl] = v

        @pl.when(c == 0)
        def _():
            wb = pltpu.async_copy(
                sred_v.at[0], degsrc_hbm.at[pl.ds(t * STRIPE, STRIPE)], sem)
            wb.wait()

        magic = jnp.full((LANES,), 0x5F3759DF, jnp.int32)
        half = jnp.full((LANES,), 0.5, jnp.float32)
        th = jnp.full((LANES,), 1.5, jnp.float32)

        @plsc.parallel_loop(0, SV, unroll=4)
        def rs_body(j):
            sl = pl.ds(j * LANES, LANES)
            v = jnp.maximum(sred_v[1, sl], ones)
            y = plsc.bitcast(magic - (plsc.bitcast(v, jnp.int32) >> 1),
                             jnp.float32)
            y = y * (th - half * v * y * y)
            y = y * (th - half * v * y * y)
            sred_v[1, sl] = y

        wb1 = pltpu.async_copy(sred_v.at[1],
                               rdst_sh.at[pl.ds(t * STRIPE, STRIPE)], sem)
        wb1.wait()
        plsc.subcore_barrier()

        rb = pltpu.async_copy(rdst_sh, degd_v, sem)
        rb.wait()

        base = c * EPH

        @plsc.parallel_loop(0, P2, unroll=8)
        def p2_body(i):
            sl = pl.ds(base + i * LANES, LANES)
            rd = plsc.load_gather(degd_v, [dst_v[sl]])
            plsc.addupdate_scatter(w_v, [src_v[sl]], rd)

        pltpu.sync_copy(w_v, wp_hbm.at[c * NS + t])

    return ew_kernel(edge_index.reshape(-1))


def _tc_head(wp, degsrc, x, W, b, Wd, bd):
    N, D = x.shape
    L = Wd.shape[1]

    def body(wp_ref, ds_ref, x_ref, W_ref, b_ref, Wd_ref, bd_ref, o_ref):
        acc = jnp.sum(wp_ref[...], axis=0, keepdims=True)
        r_src = lax.rsqrt(jnp.maximum(ds_ref[...], 1.0))
        w = (acc * r_src)[:, :N]
        t = jnp.sum(w)
        dn = (((1,), (0,)), ((), ()))
        s = lax.dot_general(w, x_ref[...], dn,
                            preferred_element_type=jnp.float32)
        pooled = lax.dot_general(s, W_ref[...], dn,
                                 preferred_element_type=jnp.float32)
        pooled = pooled + t * b_ref[...]
        logits = lax.dot_general(pooled, Wd_ref[...], dn,
                                 preferred_element_type=jnp.float32)
        logits = logits + bd_ref[...]
        e = jnp.exp(logits - jnp.max(logits))
        o_ref[...] = e / jnp.sum(e)

    return pl.pallas_call(
        body,
        out_shape=jax.ShapeDtypeStruct((1, L), jnp.float32),
    )(wp, degsrc.reshape(1, -1), x, W, b.reshape(1, D), Wd, bd.reshape(1, L))


def kernel(x, edge_index, W, b, Wd, bd):
    wp, degsrc = _sc_edge_weights(edge_index, x.shape[0])
    out = _tc_head(wp, degsrc, x, W, b, Wd, bd)
    return out.reshape(-1)
